# Initial kernel scaffold; baseline (speedup 1.0000x reference)
import functools

import jax
import jax.numpy as jnp
from jax import lax
from jax.experimental import pallas as pl
from jax.experimental.pallas import tpu as pltpu
from jax.experimental.pallas import tpu_sc as plsc

_B, _C, _H, _W = 32, 1, 512, 512
_HW = _H * _W
_NC, _NS, _L = 2, 16, 16
_CHUNK = 1024
_NV = _CHUNK // _L
_NCHUNK = _HW // _CHUNK
_ROWS_PER_CHUNK = _CHUNK // _H


def _body(x_hbm, coef_hbm, out_hbm, coef_v,
          ia, ib, ic, id_, va, vb, vc, vd,
          wx0, wx1, wy0, wy1, outv, sem):
    b = lax.axis_index("s") * _NC + lax.axis_index("c")
    bbase = b * _HW
    pltpu.sync_copy(coef_hbm.at[b], coef_v)
    ax = coef_v[0]
    bx = coef_v[1]
    cxl = coef_v[2]
    ay = coef_v[3]
    by = coef_v[4]
    cyl = coef_v[5]

    def chunk_body(cidx, carry):
        i0 = cidx * _ROWS_PER_CHUNK

        def vec_body(v, carry2):
            i_s = i0 + (v >> 5)
            j0 = (v & 31) << 4
            i_fv = jnp.full((_L,), i_s.astype(jnp.float32), dtype=jnp.float32)
            j_fv = jnp.full((_L,), j0.astype(jnp.float32), dtype=jnp.float32)
            x = ax * i_fv + (bx * j_fv + cxl)
            y = ay * i_fv + (by * j_fv + cyl)
            x0i = x.astype(jnp.int32)
            y0i = y.astype(jnp.int32)
            x0c = jnp.clip(x0i, 0, _W - 1)
            x1c = jnp.clip(x0i + 1, 0, _W - 1)
            y0c = jnp.clip(y0i, 0, _H - 1)
            y1c = jnp.clip(y0i + 1, 0, _H - 1)
            ya = (y0c << 9) + bbase
            yb = (y1c << 9) + bbase
            r = v >> 3
            c16 = (v & 7) << 4
            ia[r, pl.ds(c16, _L)] = ya + x0c
            ib[r, pl.ds(c16, _L)] = yb + x0c
            ic[r, pl.ds(c16, _L)] = ya + x1c
            id_[r, pl.ds(c16, _L)] = yb + x1c
            o = v << 4
            wx0[pl.ds(o, _L)] = x1c.astype(jnp.float32) - x
            wx1[pl.ds(o, _L)] = x - x0c.astype(jnp.float32)
            wy0[pl.ds(o, _L)] = y1c.astype(jnp.float32) - y
            wy1[pl.ds(o, _L)] = y - y0c.astype(jnp.float32)
            return carry2

        lax.fori_loop(0, _NV, vec_body, 0)

        h1 = pltpu.async_copy(x_hbm.at[ia], va, sem)
        h2 = pltpu.async_copy(x_hbm.at[ib], vb, sem)
        h3 = pltpu.async_copy(x_hbm.at[ic], vc, sem)
        h4 = pltpu.async_copy(x_hbm.at[id_], vd, sem)
        h1.wait()
        h2.wait()
        h3.wait()
        h4.wait()

        def out_body(v, carry2):
            r = v >> 3
            c16 = (v & 7) << 4
            o = v << 4
            a_v = va[r, pl.ds(c16, _L)]
            b_v = vb[r, pl.ds(c16, _L)]
            c_v = vc[r, pl.ds(c16, _L)]
            d_v = vd[r, pl.ds(c16, _L)]
            w0 = wy0[pl.ds(o, _L)]
            w1 = wy1[pl.ds(o, _L)]
            s0 = w0 * a_v + w1 * b_v
            s1 = w0 * c_v + w1 * d_v
            outv[pl.ds(o, _L)] = wx0[pl.ds(o, _L)] * s0 + wx1[pl.ds(o, _L)] * s1
            return carry2

        lax.fori_loop(0, _NV, out_body, 0)
        pltpu.sync_copy(outv, out_hbm.at[pl.ds(bbase + cidx * _CHUNK, _CHUNK)])
        return carry

    lax.fori_loop(0, _NCHUNK, chunk_body, 0)


_sc_bilinear = functools.partial(
    pl.kernel,
    out_type=jax.ShapeDtypeStruct((_B * _HW,), jnp.float32),
    mesh=plsc.VectorSubcoreMesh(
        core_axis_name="c", subcore_axis_name="s",
        num_cores=_NC, num_subcores=_NS),
    scratch_types=[
        pltpu.VMEM((6, _L), jnp.float32),
        pltpu.VMEM((_CHUNK // 128, 128), jnp.int32),
        pltpu.VMEM((_CHUNK // 128, 128), jnp.int32),
        pltpu.VMEM((_CHUNK // 128, 128), jnp.int32),
        pltpu.VMEM((_CHUNK // 128, 128), jnp.int32),
        pltpu.VMEM((_CHUNK // 128, 128), jnp.float32),
        pltpu.VMEM((_CHUNK // 128, 128), jnp.float32),
        pltpu.VMEM((_CHUNK // 128, 128), jnp.float32),
        pltpu.VMEM((_CHUNK // 128, 128), jnp.float32),
        pltpu.VMEM((_CHUNK,), jnp.float32),
        pltpu.VMEM((_CHUNK,), jnp.float32),
        pltpu.VMEM((_CHUNK,), jnp.float32),
        pltpu.VMEM((_CHUNK,), jnp.float32),
        pltpu.VMEM((_CHUNK,), jnp.float32),
        pltpu.SemaphoreType.DMA,
    ],
)(_body)


def kernel(X, transformation):
    t = transformation.reshape(_B, 2, 3).astype(jnp.float32)
    sx = jnp.float32(_W) / jnp.float32(_W - 1)
    sy = jnp.float32(_H) / jnp.float32(_H - 1)
    half_w = jnp.float32(0.5 * _W)
    lanes = jnp.arange(_L, dtype=jnp.float32)
    a_x = sx * t[:, 0, 0]
    b_x = sy * t[:, 0, 1]
    c_x = half_w * (t[:, 0, 2] - t[:, 0, 0] - t[:, 0, 1] + 1.0)
    a_y = sx * t[:, 1, 0]
    b_y = sy * t[:, 1, 1]
    c_y = half_w * (t[:, 1, 2] - t[:, 1, 0] - t[:, 1, 1] + 1.0)
    splat = lambda s: jnp.broadcast_to(s[:, None], (_B, _L))
    coef = jnp.stack([
        splat(a_x), splat(b_x), c_x[:, None] + b_x[:, None] * lanes[None, :],
        splat(a_y), splat(b_y), c_y[:, None] + b_y[:, None] * lanes[None, :],
    ], axis=1)
    out_flat = _sc_bilinear(X.reshape(_B * _HW), coef)
    return out_flat.reshape(_B, _C, _H, _W)


# speedup vs baseline: 1.3686x; 1.3686x over previous
import functools

import jax
import jax.numpy as jnp
from jax import lax
from jax.experimental import pallas as pl
from jax.experimental.pallas import tpu as pltpu
from jax.experimental.pallas import tpu_sc as plsc

_B, _C, _H, _W = 32, 1, 512, 512
_HW = _H * _W
_NC, _NS, _L = 2, 16, 16
_CHUNK = 1024
_NV = _CHUNK // _L
_NCHUNK = _HW // _CHUNK


def _body(img_hbm, xc_hbm, yc_hbm, out_hbm,
          xv, yv, ia, ib, ic, id_, va, vb, vc, vd,
          wx0, wx1, wy0, wy1, outv, sem):
    b = lax.axis_index("s") * _NC + lax.axis_index("c")
    bbase = b * _HW

    def chunk_body(cidx, carry):
        p0 = bbase + cidx * _CHUNK
        pltpu.sync_copy(xc_hbm.at[pl.ds(p0, _CHUNK)], xv)
        pltpu.sync_copy(yc_hbm.at[pl.ds(p0, _CHUNK)], yv)

        def vec_body(v, carry2):
            o = v << 4
            x = xv[pl.ds(o, _L)]
            y = yv[pl.ds(o, _L)]
            x0i = x.astype(jnp.int32)
            y0i = y.astype(jnp.int32)
            x0c = jnp.clip(x0i, 0, _W - 1)
            x1c = jnp.clip(x0i + 1, 0, _W - 1)
            y0c = jnp.clip(y0i, 0, _H - 1)
            y1c = jnp.clip(y0i + 1, 0, _H - 1)
            ya = (y0c << 9) + bbase
            yb = (y1c << 9) + bbase
            ia[pl.ds(o, _L)] = ya + x0c
            ib[pl.ds(o, _L)] = yb + x0c
            ic[pl.ds(o, _L)] = ya + x1c
            id_[pl.ds(o, _L)] = yb + x1c
            wx0[pl.ds(o, _L)] = x1c.astype(jnp.float32) - x
            wx1[pl.ds(o, _L)] = x - x0c.astype(jnp.float32)
            wy0[pl.ds(o, _L)] = y1c.astype(jnp.float32) - y
            wy1[pl.ds(o, _L)] = y - y0c.astype(jnp.float32)
            return carry2

        lax.fori_loop(0, _NV, vec_body, 0)

        h1 = pltpu.async_copy(img_hbm.at[ia], va, sem)
        h2 = pltpu.async_copy(img_hbm.at[ib], vb, sem)
        h3 = pltpu.async_copy(img_hbm.at[ic], vc, sem)
        h4 = pltpu.async_copy(img_hbm.at[id_], vd, sem)
        h1.wait()
        h2.wait()
        h3.wait()
        h4.wait()

        def out_body(v, carry2):
            o = v << 4
            a_v = va[pl.ds(o, _L)]
            b_v = vb[pl.ds(o, _L)]
            c_v = vc[pl.ds(o, _L)]
            d_v = vd[pl.ds(o, _L)]
            w0 = wy0[pl.ds(o, _L)]
            w1 = wy1[pl.ds(o, _L)]
            s0 = w0 * a_v + w1 * b_v
            s1 = w0 * c_v + w1 * d_v
            outv[pl.ds(o, _L)] = wx0[pl.ds(o, _L)] * s0 + wx1[pl.ds(o, _L)] * s1
            return carry2

        lax.fori_loop(0, _NV, out_body, 0)
        pltpu.sync_copy(outv, out_hbm.at[pl.ds(p0, _CHUNK)])
        return carry

    lax.fori_loop(0, _NCHUNK, chunk_body, 0)


_sc_bilinear = functools.partial(
    pl.kernel,
    out_type=jax.ShapeDtypeStruct((_B * _HW,), jnp.float32),
    mesh=plsc.VectorSubcoreMesh(
        core_axis_name="c", subcore_axis_name="s",
        num_cores=_NC, num_subcores=_NS),
    scratch_types=[
        pltpu.VMEM((_CHUNK,), jnp.float32),
        pltpu.VMEM((_CHUNK,), jnp.float32),
        pltpu.VMEM((_CHUNK,), jnp.int32),
        pltpu.VMEM((_CHUNK,), jnp.int32),
        pltpu.VMEM((_CHUNK,), jnp.int32),
        pltpu.VMEM((_CHUNK,), jnp.int32),
        pltpu.VMEM((_CHUNK,), jnp.float32),
        pltpu.VMEM((_CHUNK,), jnp.float32),
        pltpu.VMEM((_CHUNK,), jnp.float32),
        pltpu.VMEM((_CHUNK,), jnp.float32),
        pltpu.VMEM((_CHUNK,), jnp.float32),
        pltpu.VMEM((_CHUNK,), jnp.float32),
        pltpu.VMEM((_CHUNK,), jnp.float32),
        pltpu.VMEM((_CHUNK,), jnp.float32),
        pltpu.VMEM((_CHUNK,), jnp.float32),
        pltpu.SemaphoreType.DMA,
    ],
)(_body)


def _make_grids():
    x_linspace = jnp.linspace(-1.0, 1.0, _W)
    y_linspace = jnp.linspace(-1.0, 1.0, _H)
    x_c, y_c = jnp.meshgrid(x_linspace, y_linspace, indexing='ij')
    ones = jnp.ones_like(x_c.reshape(-1))
    grid = jnp.concatenate([x_c.reshape(-1), y_c.reshape(-1), ones], axis=0)
    grids = jnp.tile(grid.reshape(-1), (_B,))
    return grids.reshape(_B, 3, _HW)


def kernel(X, transformation):
    transformations = transformation.reshape(_B, 2, 3)
    grids = _make_grids().astype(jnp.float32)
    sg = jnp.einsum('bij,bjk->bik', transformations, grids)
    x = 0.5 * (sg[:, 0, :].reshape(-1).astype(jnp.float32) + 1.0) * _W
    y = 0.5 * (sg[:, 1, :].reshape(-1).astype(jnp.float32) + 1.0) * _H
    out_flat = _sc_bilinear(X.reshape(_B * _HW), x, y)
    return out_flat.reshape(_B, _C, _H, _W)


# speedup vs baseline: 1.4669x; 1.0718x over previous
import functools

import jax
import jax.numpy as jnp
from jax import lax
from jax.experimental import pallas as pl
from jax.experimental.pallas import tpu as pltpu
from jax.experimental.pallas import tpu_sc as plsc

_B, _C, _H, _W = 32, 1, 512, 512
_HW = _H * _W
_NC, _NS, _L = 2, 16, 16
_CHUNK = 1024
_NV = _CHUNK // _L
_NCHUNK = _HW // _CHUNK


def _body(img_hbm, xc_hbm, yc_hbm, out_hbm, *scr):
    slots = (scr[0:15], scr[15:30])
    sem_g = scr[30:32]
    sem_xy = scr[32:34]
    sem_o = scr[34:36]
    b = lax.axis_index("s") * _NC + lax.axis_index("c")
    bbase = b * _HW

    def fire_xy(c, s):
        xv, yv = slots[s][0], slots[s][1]
        off = bbase + c * _CHUNK
        pltpu.async_copy(xc_hbm.at[pl.ds(off, _CHUNK)], xv, sem_xy[s])
        pltpu.async_copy(yc_hbm.at[pl.ds(off, _CHUNK)], yv, sem_xy[s])

    def wait_xy(s):
        xv, yv = slots[s][0], slots[s][1]
        pltpu.make_async_copy(xc_hbm.at[pl.ds(bbase, _CHUNK)], xv, sem_xy[s]).wait()
        pltpu.make_async_copy(yc_hbm.at[pl.ds(bbase, _CHUNK)], yv, sem_xy[s]).wait()

    def stage(s):
        xv, yv, ia, ib, ic2, id2 = slots[s][0:6]
        wx0, wx1, wy0, wy1 = slots[s][10:14]

        def vec_body(v, carry):
            o = v << 4
            x = xv[pl.ds(o, _L)]
            y = yv[pl.ds(o, _L)]
            x0i = x.astype(jnp.int32)
            y0i = y.astype(jnp.int32)
            x0c = jnp.clip(x0i, 0, _W - 1)
            x1c = jnp.clip(x0i + 1, 0, _W - 1)
            y0c = jnp.clip(y0i, 0, _H - 1)
            y1c = jnp.clip(y0i + 1, 0, _H - 1)
            ya = (y0c << 9) + bbase
            yb = (y1c << 9) + bbase
            ia[pl.ds(o, _L)] = ya + x0c
            ib[pl.ds(o, _L)] = yb + x0c
            ic2[pl.ds(o, _L)] = ya + x1c
            id2[pl.ds(o, _L)] = yb + x1c
            wx0[pl.ds(o, _L)] = x1c.astype(jnp.float32) - x
            wx1[pl.ds(o, _L)] = x - x0c.astype(jnp.float32)
            wy0[pl.ds(o, _L)] = y1c.astype(jnp.float32) - y
            wy1[pl.ds(o, _L)] = y - y0c.astype(jnp.float32)
            return carry

        lax.fori_loop(0, _NV, vec_body, 0)

    def fire_gather(s):
        ia, ib, ic2, id2, va, vb2, vc2, vd2 = slots[s][2:10]
        pltpu.async_copy(img_hbm.at[ia], va, sem_g[s])
        pltpu.async_copy(img_hbm.at[ib], vb2, sem_g[s])
        pltpu.async_copy(img_hbm.at[ic2], vc2, sem_g[s])
        pltpu.async_copy(img_hbm.at[id2], vd2, sem_g[s])

    def wait_gather(s):
        ia, ib, ic2, id2, va, vb2, vc2, vd2 = slots[s][2:10]
        pltpu.make_async_copy(img_hbm.at[ia], va, sem_g[s]).wait()
        pltpu.make_async_copy(img_hbm.at[ib], vb2, sem_g[s]).wait()
        pltpu.make_async_copy(img_hbm.at[ic2], vc2, sem_g[s]).wait()
        pltpu.make_async_copy(img_hbm.at[id2], vd2, sem_g[s]).wait()

    def compute_out(c, s):
        va, vb2, vc2, vd2, wx0, wx1, wy0, wy1, ov = slots[s][6:15]

        def out_body(v, carry):
            o = v << 4
            a_v = va[pl.ds(o, _L)]
            b_v = vb2[pl.ds(o, _L)]
            c_v = vc2[pl.ds(o, _L)]
            d_v = vd2[pl.ds(o, _L)]
            w0 = wy0[pl.ds(o, _L)]
            w1 = wy1[pl.ds(o, _L)]
            s0 = w0 * a_v + w1 * b_v
            s1 = w0 * c_v + w1 * d_v
            ov[pl.ds(o, _L)] = wx0[pl.ds(o, _L)] * s0 + wx1[pl.ds(o, _L)] * s1
            return carry

        lax.fori_loop(0, _NV, out_body, 0)
        pltpu.async_copy(ov, out_hbm.at[pl.ds(bbase + c * _CHUNK, _CHUNK)],
                         sem_o[s])

    def wait_out(s):
        ov = slots[s][14]
        pltpu.make_async_copy(ov, out_hbm.at[pl.ds(bbase, _CHUNK)],
                              sem_o[s]).wait()

    for c0 in (0, 1):
        fire_xy(c0, c0)
        wait_xy(c0)
        stage(c0)
        fire_xy(c0 + 2, c0)
        fire_gather(c0)

    def pair_body(g, carry):
        for s in (0, 1):
            c = 2 * g + s
            wait_gather(s)

            @pl.when(g > 0)
            def _():
                wait_out(s)

            compute_out(c, s)
            wait_xy(s)
            stage(s)
            fire_xy((c + 4) & (_NCHUNK - 1), s)
            fire_gather(s)
        return carry

    lax.fori_loop(0, _NCHUNK // 2, pair_body, 0)

    for s in (0, 1):
        wait_out(s)
        wait_gather(s)
        wait_xy(s)


_sc_bilinear = functools.partial(
    pl.kernel,
    out_type=jax.ShapeDtypeStruct((_B * _HW,), jnp.float32),
    mesh=plsc.VectorSubcoreMesh(
        core_axis_name="c", subcore_axis_name="s",
        num_cores=_NC, num_subcores=_NS),
    scratch_types=(
        ([pltpu.VMEM((_CHUNK,), jnp.float32)] * 2 +
         [pltpu.VMEM((_CHUNK,), jnp.int32)] * 4 +
         [pltpu.VMEM((_CHUNK,), jnp.float32)] * 9) * 2 +
        [pltpu.SemaphoreType.DMA] * 6
    ),
)(_body)


def _make_grids():
    x_linspace = jnp.linspace(-1.0, 1.0, _W)
    y_linspace = jnp.linspace(-1.0, 1.0, _H)
    x_c, y_c = jnp.meshgrid(x_linspace, y_linspace, indexing='ij')
    ones = jnp.ones_like(x_c.reshape(-1))
    grid = jnp.concatenate([x_c.reshape(-1), y_c.reshape(-1), ones], axis=0)
    grids = jnp.tile(grid.reshape(-1), (_B,))
    return grids.reshape(_B, 3, _HW)


def kernel(X, transformation):
    transformations = transformation.reshape(_B, 2, 3)
    grids = _make_grids().astype(jnp.float32)
    sg = jnp.einsum('bij,bjk->bik', transformations, grids)
    x = 0.5 * (sg[:, 0, :].reshape(-1).astype(jnp.float32) + 1.0) * _W
    y = 0.5 * (sg[:, 1, :].reshape(-1).astype(jnp.float32) + 1.0) * _H
    out_flat = _sc_bilinear(X.reshape(_B * _HW), x, y)
    return out_flat.reshape(_B, _C, _H, _W)


# speedup vs baseline: 1.9225x; 1.3106x over previous
import functools

import jax
import jax.numpy as jnp
from jax import lax
from jax.experimental import pallas as pl
from jax.experimental.pallas import tpu as pltpu
from jax.experimental.pallas import tpu_sc as plsc

_B, _C, _H, _W = 32, 1, 512, 512
_HW = _H * _W
_N = _B * _HW
_NC, _NS, _L = 2, 16, 16
_CHUNK = 1024
_NV = _CHUNK // _L
_NCHUNK = _HW // _CHUNK


def _body(img_hbm, xc_hbm, yc_hbm, out_hbm, *scr):
    slots = (scr[0:15], scr[15:30])
    zbuf = scr[30]
    flags = scr[31]
    sem_g = scr[32:34]
    sem_xy = scr[34:36]
    sem_o = scr[36:38]
    b = lax.axis_index("s") * _NC + lax.axis_index("c")
    bbase = b * _HW
    zero16 = jnp.zeros((_L,), jnp.float32)

    def zinit(v, carry):
        zbuf[pl.ds(v << 4, _L)] = zero16
        return carry

    lax.fori_loop(0, _NV, zinit, 0)

    def fire_xy(c, s):
        xv, yv = slots[s][0], slots[s][1]
        off = bbase + c * _CHUNK
        pltpu.async_copy(xc_hbm.at[pl.ds(off, _CHUNK)], xv, sem_xy[s])
        pltpu.async_copy(yc_hbm.at[pl.ds(off, _CHUNK)], yv, sem_xy[s])

    def wait_xy(s):
        xv, yv = slots[s][0], slots[s][1]
        pltpu.make_async_copy(xc_hbm.at[pl.ds(bbase, _CHUNK)], xv, sem_xy[s]).wait()
        pltpu.make_async_copy(yc_hbm.at[pl.ds(bbase, _CHUNK)], yv, sem_xy[s]).wait()

    def stage(s):
        xv, yv, ia, ib, ic2, id2 = slots[s][0:6]
        wx0, wx1, wy0, wy1 = slots[s][10:14]

        def vec_body(v, vmax):
            o = v << 4
            x = xv[pl.ds(o, _L)]
            y = yv[pl.ds(o, _L)]
            x0i = x.astype(jnp.int32)
            y0i = y.astype(jnp.int32)
            x0c = jnp.clip(x0i, 0, _W - 1)
            x1c = jnp.clip(x0i + 1, 0, _W - 1)
            y0c = jnp.clip(y0i, 0, _H - 1)
            y1c = jnp.clip(y0i + 1, 0, _H - 1)
            ya = (y0c << 9) + bbase
            yb = (y1c << 9) + bbase
            ia[pl.ds(o, _L)] = ya + x0c
            ib[pl.ds(o, _L)] = yb + x0c
            ic2[pl.ds(o, _L)] = ya + x1c
            id2[pl.ds(o, _L)] = yb + x1c
            x0f = x0c.astype(jnp.float32)
            x1f = x1c.astype(jnp.float32)
            y0f = y0c.astype(jnp.float32)
            y1f = y1c.astype(jnp.float32)
            wx0[pl.ds(o, _L)] = x1f - x
            wx1[pl.ds(o, _L)] = x - x0f
            wy0[pl.ds(o, _L)] = y1f - y
            wy1[pl.ds(o, _L)] = y - y0f
            return jnp.maximum(vmax, (x1f - x0f) * (y1f - y0f))

        vmax = lax.fori_loop(0, _NV, vec_body, zero16)
        flags[s] = (jnp.max(vmax, axis=0) > 0.0).astype(jnp.int32)

    def fire_gather(s):
        ia, ib, ic2, id2, va, vb2, vc2, vd2 = slots[s][2:10]
        pltpu.async_copy(img_hbm.at[ia], va, sem_g[s])
        pltpu.async_copy(img_hbm.at[ib], vb2, sem_g[s])
        pltpu.async_copy(img_hbm.at[ic2], vc2, sem_g[s])
        pltpu.async_copy(img_hbm.at[id2], vd2, sem_g[s])

    def wait_gather(s):
        ia, ib, ic2, id2, va, vb2, vc2, vd2 = slots[s][2:10]
        pltpu.make_async_copy(img_hbm.at[ia], va, sem_g[s]).wait()
        pltpu.make_async_copy(img_hbm.at[ib], vb2, sem_g[s]).wait()
        pltpu.make_async_copy(img_hbm.at[ic2], vc2, sem_g[s]).wait()
        pltpu.make_async_copy(img_hbm.at[id2], vd2, sem_g[s]).wait()

    def compute_out(c, s):
        va, vb2, vc2, vd2, wx0, wx1, wy0, wy1, ov = slots[s][6:15]

        def out_body(v, carry):
            o = v << 4
            a_v = va[pl.ds(o, _L)]
            b_v = vb2[pl.ds(o, _L)]
            c_v = vc2[pl.ds(o, _L)]
            d_v = vd2[pl.ds(o, _L)]
            w0 = wy0[pl.ds(o, _L)]
            w1 = wy1[pl.ds(o, _L)]
            s0 = w0 * a_v + w1 * b_v
            s1 = w0 * c_v + w1 * d_v
            ov[pl.ds(o, _L)] = wx0[pl.ds(o, _L)] * s0 + wx1[pl.ds(o, _L)] * s1
            return carry

        lax.fori_loop(0, _NV, out_body, 0)
        pltpu.async_copy(ov, out_hbm.at[pl.ds(bbase + c * _CHUNK, _CHUNK)],
                         sem_o[s])

    def fire_zero_out(c, s):
        pltpu.async_copy(zbuf, out_hbm.at[pl.ds(bbase + c * _CHUNK, _CHUNK)],
                         sem_o[s])

    def wait_out(s):
        ov = slots[s][14]
        pltpu.make_async_copy(ov, out_hbm.at[pl.ds(bbase, _CHUNK)],
                              sem_o[s]).wait()

    for c0 in (0, 1):
        fire_xy(c0, c0)
        wait_xy(c0)
        stage(c0)

        @pl.when(flags[c0] > 0)
        def _():
            fire_gather(c0)

        fire_xy(c0 + 2, c0)

    def pair_body(g, carry):
        for s in (0, 1):
            c = 2 * g + s
            f_old = flags[s]

            @pl.when(f_old > 0)
            def _():
                wait_gather(s)

            @pl.when(g > 0)
            def _():
                wait_out(s)

            @pl.when(f_old > 0)
            def _():
                compute_out(c, s)

            @pl.when(f_old == 0)
            def _():
                fire_zero_out(c, s)

            wait_xy(s)
            stage(s)
            fire_xy((c + 4) & (_NCHUNK - 1), s)

            @pl.when(flags[s] > 0)
            def _():
                fire_gather(s)

        return carry

    lax.fori_loop(0, _NCHUNK // 2, pair_body, 0)

    for s in (0, 1):
        wait_out(s)

        @pl.when(flags[s] > 0)
        def _():
            wait_gather(s)

        wait_xy(s)


_sc_bilinear = functools.partial(
    pl.kernel,
    out_type=jax.ShapeDtypeStruct((_N,), jnp.float32),
    mesh=plsc.VectorSubcoreMesh(
        core_axis_name="c", subcore_axis_name="s",
        num_cores=_NC, num_subcores=_NS),
    compiler_params=pltpu.CompilerParams(needs_layout_passes=False),
    scratch_types=(
        ([pltpu.VMEM((_CHUNK,), jnp.float32)] * 2 +
         [pltpu.VMEM((_CHUNK,), jnp.int32)] * 4 +
         [pltpu.VMEM((_CHUNK,), jnp.float32)] * 9) * 2 +
        [pltpu.VMEM((_CHUNK,), jnp.float32)] +
        [pltpu.SMEM((2,), jnp.int32)] +
        [pltpu.SemaphoreType.DMA] * 6
    ),
)(_body)


def _make_grids():
    x_linspace = jnp.linspace(-1.0, 1.0, _W)
    y_linspace = jnp.linspace(-1.0, 1.0, _H)
    x_c, y_c = jnp.meshgrid(x_linspace, y_linspace, indexing='ij')
    ones = jnp.ones_like(x_c.reshape(-1))
    grid = jnp.concatenate([x_c.reshape(-1), y_c.reshape(-1), ones], axis=0)
    grids = jnp.tile(grid.reshape(-1), (_B,))
    return grids.reshape(_B, 3, _HW)


def kernel(X, transformation):
    transformations = transformation.reshape(_B, 2, 3)
    grids = _make_grids().astype(jnp.float32)
    sg = jnp.einsum('bij,bjk->bik', transformations, grids)
    x = 0.5 * (sg[:, 0, :].reshape(-1).astype(jnp.float32) + 1.0) * _W
    y = 0.5 * (sg[:, 1, :].reshape(-1).astype(jnp.float32) + 1.0) * _H
    out_flat = _sc_bilinear(X.reshape(_N), x, y)
    return out_flat.reshape(_B, _C, _H, _W)
